# baseline (speedup 1.0000x reference)
import functools

import jax
import jax.numpy as jnp
from jax import lax
from jax.experimental import pallas as pl
from jax.experimental.pallas import tpu as pltpu
from jax.experimental.pallas import tpu_sc as plsc

N = 10000
E = 160000
NP = 10240
EP = 165888
NC = 2
NS = 16
CH = 128
f32 = jnp.float32
i32 = jnp.int32

_MESH = dict(core_axis_name="c", subcore_axis_name="s", num_cores=NC,
             num_subcores=NS)
_SC_PARAMS = pltpu.CompilerParams(needs_layout_passes=False)


def _k1_body(e0, e1, ee0, ee1, wrel, wroot, wo, ust, tst, q0, q1):
    a0 = e0[...]
    a1 = e1[...]
    wr = wrel[...]
    wt = wroot[...]
    for c in range(2):
        cols = slice(c * 128, (c + 1) * 128)
        base = c * 2000
        ust[base:base + 1000] = jnp.dot(a0, wr[0:128, cols],
                                        preferred_element_type=f32)
        ust[base + 1000:base + 2000] = jnp.dot(a1, wr[128:256, cols],
                                               preferred_element_type=f32)
        tst[base:base + 1000] = jnp.dot(a0, wt[0:128, cols],
                                        preferred_element_type=f32)
        tst[base + 1000:base + 2000] = jnp.dot(a1, wt[128:256, cols],
                                               preferred_element_type=f32)
    w = wo[...]
    q0[...] = jnp.dot(ee0[...], w[512:640, :], preferred_element_type=f32)
    q1[...] = jnp.dot(ee1[...], w[640:768, :], preferred_element_type=f32)


_k1 = pl.pallas_call(
    _k1_body,
    out_shape=[
        jax.ShapeDtypeStruct((4000, 128), f32),
        jax.ShapeDtypeStruct((4000, 128), f32),
        jax.ShapeDtypeStruct((100, 4), f32),
        jax.ShapeDtypeStruct((100, 4), f32),
    ],
)


def _copy128(dst_ref, src_ref, base):
    for j in range(8):
        dst_ref[pl.ds(j * 16, 16)] = src_ref[pl.ds(base + j * 16, 16)]


def _add_into(a, b):
    def row(r, _):
        for j in range(8):
            sl = pl.ds(j * 16, 16)
            a[r, sl] = a[r, sl] + b[r, sl]
        return 0

    lax.fori_loop(0, CH, row, 0)


def _k2_body(ust, tst, x0h, x1h, dst, yst, rst, deg_out,
             i0f, i1f, i0c, i1c, bufa, bufb, bufc, bufd, didx_all, degbuf,
             rbuf, resbuf, deg_sh, sema, semb, semc, semd):
    c = lax.axis_index("c")
    s = lax.axis_index("s")
    tbase = c * 2000
    z16 = jnp.zeros((16,), f32)
    ones16 = jnp.ones((16,), f32)

    pltpu.sync_copy(x0h.at[pl.ds(s * 640, 640)], i0f)
    pltpu.sync_copy(x1h.at[pl.ds(s * 640, 640)], i1f)
    for r in range(40):
        sl = pl.ds(r * 16, 16)
        i0f[sl] = i0f[sl] + tbase
        i1f[sl] = i1f[sl] + (tbase + 1000)

    for k in range(5):
        off = s * 640 + k * CH
        _copy128(i0c, i0f, k * CH)
        _copy128(i1c, i1f, k * CH)
        du = pltpu.async_copy(ust.at[i0c], bufa, sema)
        dv = pltpu.async_copy(ust.at[i1c], bufb, semb)
        dt = pltpu.async_copy(tst.at[i0c], bufc, semc)
        dw = pltpu.async_copy(tst.at[i1c], bufd, semd)
        du.wait()
        dv.wait()
        _add_into(bufa, bufb)
        pltpu.sync_copy(bufa, yst.at[pl.ds(c * NP + off, CH)])
        dt.wait()
        dw.wait()
        _add_into(bufc, bufd)
        pltpu.sync_copy(bufc, rst.at[pl.ds(c * NP + off, CH)])

    @pl.when(c == 0)
    def _():
        def zdeg(q, _):
            degbuf[pl.ds(q * 16, 16)] = z16
            return 0

        lax.fori_loop(0, 1024, zdeg, 0)
        pltpu.sync_copy(dst.at[pl.ds(s * (EP // NS), EP // NS)], didx_all)

        def echunk(k2, _):
            for j in range(8):
                dj = didx_all[pl.ds(k2 * CH + j * 16, 16)]
                plsc.addupdate_scatter(degbuf, [dj], ones16)
            return 0

        lax.fori_loop(0, _ECH, echunk, 0)
        pltpu.sync_copy(degbuf, deg_sh.at[s])
        plsc.subcore_barrier()
        for p in range(NS):
            pltpu.sync_copy(deg_sh.at[p, pl.ds(s * 1024, 1024)], rbuf.at[p])

        def red(g, _):
            sl = pl.ds(g * 16, 16)
            v = rbuf[0, sl]
            for p in range(1, NS):
                v = v + rbuf[p, sl]
            resbuf[sl] = v
            return 0

        lax.fori_loop(0, 64, red, 0)
        pltpu.sync_copy(resbuf, deg_out.at[pl.ds(s * 1024, 1024)])


_k2 = functools.partial(
    pl.kernel,
    out_type=[
        jax.ShapeDtypeStruct((2 * NP, 128), f32),
        jax.ShapeDtypeStruct((2 * NP, 128), f32),
        jax.ShapeDtypeStruct((16384,), f32),
    ],
    mesh=plsc.VectorSubcoreMesh(**_MESH),
    compiler_params=_SC_PARAMS,
    scratch_types=[
        pltpu.VMEM((640,), i32),
        pltpu.VMEM((640,), i32),
        pltpu.VMEM((CH,), i32),
        pltpu.VMEM((CH,), i32),
        pltpu.VMEM((CH, 128), f32),
        pltpu.VMEM((CH, 128), f32),
        pltpu.VMEM((CH, 128), f32),
        pltpu.VMEM((CH, 128), f32),
        pltpu.VMEM((EP // NS,), i32),
        pltpu.VMEM((16384,), f32),
        pltpu.VMEM((NS, 1024), f32),
        pltpu.VMEM((1024,), f32),
        pltpu.VMEM_SHARED((NS, 16384), f32),
        pltpu.SemaphoreType.DMA,
        pltpu.SemaphoreType.DMA,
        pltpu.SemaphoreType.DMA,
        pltpu.SemaphoreType.DMA,
    ],
)(_k2_body)


_ECH = EP // NS // CH
_MCH = 64
_MNC = EP // NS // _MCH


def _copy64(dst_ref, src_ref, base):
    for j in range(4):
        dst_ref[pl.ds(j * 16, 16)] = src_ref[pl.ds(base + j * 16, 16)]


def _msg_body(table, src, dst, acc_out,
              sidx_flat, didx_flat, si0, si1, si2, di0, di1, di2,
              rows0, rows1, rows2, zbuf, acc,
              g0, g1, g2, s0, s1, s2):
    c = lax.axis_index("c")
    s = lax.axis_index("s")
    tab_off = c * NP
    epw = EP // NS
    z16 = jnp.zeros((16,), f32)
    sis = (si0, si1, si2)
    dis = (di0, di1, di2)
    rows = (rows0, rows1, rows2)
    gsem = (g0, g1, g2)
    ssem = (s0, s1, s2)

    for i in range(8):
        for j in range(8):
            zbuf[i, pl.ds(j * 16, 16)] = z16

    def zacc(q, _):
        pltpu.sync_copy(zbuf, acc.at[pl.ds(s * 640 + q * 8, 8)])
        return 0

    lax.fori_loop(0, 80, zacc, 0)

    pltpu.sync_copy(src.at[pl.ds(s * epw, epw)], sidx_flat)
    pltpu.sync_copy(dst.at[pl.ds(s * epw, epw)], didx_flat)

    def addoff(r, _):
        sl = pl.ds(r * 16, 16)
        sidx_flat[sl] = sidx_flat[sl] + tab_off
        return 0

    lax.fori_loop(0, epw // 16, addoff, 0)
    plsc.subcore_barrier()

    def fire_gather(b, k):
        _copy64(sis[b], sidx_flat, k * _MCH)
        pltpu.async_copy(table.at[sis[b]], rows[b], gsem[b])

    def wait_gather(b):
        pltpu.make_async_copy(table.at[sis[b]], rows[b], gsem[b]).wait()

    def fire_scatter(b, k):
        _copy64(dis[b], didx_flat, k * _MCH)
        pltpu.async_copy(rows[b], acc.at[dis[b]], ssem[b], add=True)

    def wait_scatter(b):
        pltpu.make_async_copy(rows[b], acc.at[dis[b]], ssem[b]).wait()

    fire_gather(0, 0)
    fire_gather(1, 1)

    def rnd(r, _):
        for b in range(3):
            k = 3 * r + b
            b2 = (b + 2) % 3
            wait_gather(b)
            fire_scatter(b, k)

            @pl.when(k >= 1)
            def _():
                wait_scatter(b2)

            @pl.when(k + 2 < _MNC)
            def _():
                fire_gather(b2, k + 2)
        return 0

    lax.fori_loop(0, _MNC // 3, rnd, 0)
    wait_scatter((_MNC - 1) % 3)

    plsc.subcore_barrier()
    pltpu.sync_copy(acc.at[pl.ds(s * 640, 640)],
                    acc_out.at[pl.ds(tab_off + s * 640, 640)])


_kmsg = functools.partial(
    pl.kernel,
    out_type=jax.ShapeDtypeStruct((2 * NP, 128), f32),
    mesh=plsc.VectorSubcoreMesh(**_MESH),
    compiler_params=_SC_PARAMS,
    scratch_types=[
        pltpu.VMEM((EP // NS,), i32),
        pltpu.VMEM((EP // NS,), i32),
        pltpu.VMEM((_MCH,), i32),
        pltpu.VMEM((_MCH,), i32),
        pltpu.VMEM((_MCH,), i32),
        pltpu.VMEM((_MCH,), i32),
        pltpu.VMEM((_MCH,), i32),
        pltpu.VMEM((_MCH,), i32),
        pltpu.VMEM((_MCH, 128), f32),
        pltpu.VMEM((_MCH, 128), f32),
        pltpu.VMEM((_MCH, 128), f32),
        pltpu.VMEM((8, 128), f32),
        pltpu.VMEM_SHARED((NP, 128), f32),
        pltpu.SemaphoreType.DMA,
        pltpu.SemaphoreType.DMA,
        pltpu.SemaphoreType.DMA,
        pltpu.SemaphoreType.DMA,
        pltpu.SemaphoreType.DMA,
        pltpu.SemaphoreType.DMA,
    ],
)(_msg_body)


def _k4_body(a_ref, r_ref, d_ref, wrel, wroot, b0_ref, z_ref, xr_ref):
    a = jnp.concatenate([a_ref[0], a_ref[1]], axis=1)
    r = jnp.concatenate([r_ref[0], r_ref[1]], axis=1)
    inv = 1.0 / jnp.maximum(d_ref[...], 1.0)
    pre = a * inv + r + b0_ref[...]
    x1 = jnp.where(pre > 0, pre, jnp.exp(pre) - 1.0)
    z = jnp.dot(x1, wrel[...], preferred_element_type=f32)
    z_ref[0] = z[:, :128]
    z_ref[1] = z[:, 128:]
    xr_ref[...] = jnp.dot(x1, wroot[...], preferred_element_type=f32)


_BM = 256
_k4 = pl.pallas_call(
    _k4_body,
    grid=(NP // _BM,),
    in_specs=[
        pl.BlockSpec((2, _BM, 128), lambda i: (0, i, 0)),
        pl.BlockSpec((2, _BM, 128), lambda i: (0, i, 0)),
        pl.BlockSpec((_BM, 1), lambda i: (i, 0)),
        pl.BlockSpec((256, 256), lambda i: (0, 0)),
        pl.BlockSpec((256, 256), lambda i: (0, 0)),
        pl.BlockSpec((1, 256), lambda i: (0, 0)),
    ],
    out_specs=[
        pl.BlockSpec((2, _BM, 128), lambda i: (0, i, 0)),
        pl.BlockSpec((_BM, 256), lambda i: (i, 0)),
    ],
    out_shape=[
        jax.ShapeDtypeStruct((2, NP, 128), f32),
        jax.ShapeDtypeStruct((NP, 256), f32),
    ],
)


def _k6_body(a_ref, xr_ref, d_ref, wcat, b1_ref, p_ref):
    a = jnp.concatenate([a_ref[0], a_ref[1]], axis=1)
    inv = 1.0 / jnp.maximum(d_ref[...], 1.0)
    pre = a * inv + xr_ref[...] + b1_ref[...]
    x2 = jnp.where(pre > 0, pre, jnp.exp(pre) - 1.0)
    p_ref[...] = jnp.dot(x2, wcat[...], preferred_element_type=f32)


_k6 = pl.pallas_call(
    _k6_body,
    grid=(NP // _BM,),
    in_specs=[
        pl.BlockSpec((2, _BM, 128), lambda i: (0, i, 0)),
        pl.BlockSpec((_BM, 256), lambda i: (i, 0)),
        pl.BlockSpec((_BM, 1), lambda i: (i, 0)),
        pl.BlockSpec((256, 8), lambda i: (0, 0)),
        pl.BlockSpec((1, 256), lambda i: (0, 0)),
    ],
    out_specs=pl.BlockSpec((_BM, 8), lambda i: (i, 0)),
    out_shape=jax.ShapeDtypeStruct((NP, 8), f32),
)


def _k7_body(st_ref, w_ref, bo_ref, s_ref):
    s_ref[...] = (jnp.dot(st_ref[...], w_ref[...],
                          preferred_element_type=f32) + bo_ref[...])


_SBM = 2000
_k7 = pl.pallas_call(
    _k7_body,
    grid=(E // _SBM,),
    in_specs=[
        pl.BlockSpec((_SBM, 16), lambda i: (i, 0)),
        pl.BlockSpec((16, 4), lambda i: (0, 0)),
        pl.BlockSpec((1, 4), lambda i: (0, 0)),
    ],
    out_specs=pl.BlockSpec((_SBM, 4), lambda i: (i, 0)),
    out_shape=jax.ShapeDtypeStruct((EP, 4), f32),
)


def _k8_body(p_hbm, q0_hbm, q1_hbm, s_hbm, src, dst, ea0, ea1, out,
             pbuf, q0b, q1b, sbuf, obuf, sia, dia, e0a, e1a):
    c = lax.axis_index("c")
    s = lax.axis_index("s")
    w = s * NC + c
    per_w = EP // (NC * NS)
    half = per_w // 2
    pltpu.sync_copy(p_hbm, pbuf)
    pltpu.sync_copy(q0_hbm, q0b)
    pltpu.sync_copy(q1_hbm, q1b)
    pltpu.sync_copy(src.at[pl.ds(w * per_w, per_w)], sia)
    pltpu.sync_copy(dst.at[pl.ds(w * per_w, per_w)], dia)
    pltpu.sync_copy(ea0.at[pl.ds(w * per_w, per_w)], e0a)
    pltpu.sync_copy(ea1.at[pl.ds(w * per_w, per_w)], e1a)
    iota16 = lax.iota(i32, 16)
    div4 = iota16 // 4
    mod4 = iota16 % 4

    for h in range(2):
        base = w * per_w + h * half
        pltpu.sync_copy(s_hbm.at[pl.ds(base * 4, half * 4)], sbuf)

        def group(g, _):
            for u in range(4):
                e = (h * half) + (g * 16 + u * 4) + div4
                sj = plsc.load_gather(sia, [e])
                dj = plsc.load_gather(dia, [e])
                a0 = plsc.load_gather(e0a, [e])
                a1 = plsc.load_gather(e1a, [e])
                sl = pl.ds(g * 64 + u * 16, 16)
                v = sbuf[sl]
                v = v + plsc.load_gather(pbuf, [sj * 8 + mod4])
                v = v + plsc.load_gather(pbuf, [dj * 8 + (mod4 + 4)])
                v = v + plsc.load_gather(q0b, [a0 * 4 + mod4])
                v = v + plsc.load_gather(q1b, [a1 * 4 + mod4])
                obuf[sl] = v
            return 0

        lax.fori_loop(0, half // 16, group, 0)
        pltpu.sync_copy(obuf, out.at[pl.ds(base * 4, half * 4)])


_k8 = functools.partial(
    pl.kernel,
    out_type=jax.ShapeDtypeStruct((EP * 4,), f32),
    mesh=plsc.VectorSubcoreMesh(**_MESH),
    compiler_params=_SC_PARAMS,
    scratch_types=[
        pltpu.VMEM((NP * 8,), f32),
        pltpu.VMEM((400,), f32),
        pltpu.VMEM((400,), f32),
        pltpu.VMEM((EP // 32 // 2 * 4,), f32),
        pltpu.VMEM((EP // 32 // 2 * 4,), f32),
        pltpu.VMEM((EP // 32,), i32),
        pltpu.VMEM((EP // 32,), i32),
        pltpu.VMEM((EP // 32,), i32),
        pltpu.VMEM((EP // 32,), i32),
    ],
)(_k8_body)


def kernel(x, edge_index, edge_attr, static_edge_features,
           node_emb_0, node_emb_1, edge_emb_0, edge_emb_1,
           W_rel0, W_root0, b0, W_rel1, W_root1, b1, Wo, bo):
    x0 = jnp.pad(x[:, 0].astype(i32), (0, NP - N))
    x1 = jnp.pad(x[:, 1].astype(i32), (0, NP - N))
    src = jnp.pad(edge_index[0].astype(i32), (0, EP - E))
    dst = jnp.pad(edge_index[1].astype(i32), (0, EP - E), constant_values=N)
    ea0 = jnp.pad(edge_attr[:, 0].astype(i32), (0, EP - E))
    ea1 = jnp.pad(edge_attr[:, 1].astype(i32), (0, EP - E))
    ust, tst, q0, q1 = _k1(node_emb_0, node_emb_1, edge_emb_0, edge_emb_1,
                           W_rel0, W_root0, Wo)
    yst, rst, deg = _k2(ust, tst, x0, x1, dst)
    acc1 = _kmsg(yst, src, dst)
    deg2 = deg[:NP].reshape(NP, 1)
    z3, xr1 = _k4(acc1.reshape(2, NP, 128), rst.reshape(2, NP, 128), deg2,
                  W_rel1, W_root1, b0.reshape(1, 256))
    acc2 = _kmsg(z3.reshape(2 * NP, 128), src, dst)
    wcat = jnp.concatenate([Wo[0:256], Wo[256:512]], axis=1)
    p = _k6(acc2.reshape(2, NP, 128), xr1, deg2, wcat, b1.reshape(1, 256))
    s_edges = _k7(static_edge_features, Wo[768:784],
                  bo.reshape(1, 4))
    out = _k8(p.reshape(NP * 8), q0.reshape(400), q1.reshape(400),
              s_edges.reshape(EP * 4), src, dst, ea0, ea1)
    return out.reshape(EP, 4)[:E]


# speedup vs baseline: 1.1672x; 1.1672x over previous
import functools

import jax
import jax.numpy as jnp
from jax import lax
from jax.experimental import pallas as pl
from jax.experimental.pallas import tpu as pltpu
from jax.experimental.pallas import tpu_sc as plsc

N = 10000
E = 160000
NP = 10240
EP = 163840
NC = 2
NS = 16
CH = 128
f32 = jnp.float32
i32 = jnp.int32

_MESH = dict(core_axis_name="c", subcore_axis_name="s", num_cores=NC,
             num_subcores=NS)
_SC_PARAMS = pltpu.CompilerParams(needs_layout_passes=False)


def _k1_body(e0, e1, ee0, ee1, wrel, wroot, wo, ust, tst, q0, q1):
    a0 = e0[...]
    a1 = e1[...]
    wr = wrel[...]
    wt = wroot[...]
    for c in range(2):
        cols = slice(c * 128, (c + 1) * 128)
        base = c * 2000
        ust[base:base + 1000] = jnp.dot(a0, wr[0:128, cols],
                                        preferred_element_type=f32)
        ust[base + 1000:base + 2000] = jnp.dot(a1, wr[128:256, cols],
                                               preferred_element_type=f32)
        tst[base:base + 1000] = jnp.dot(a0, wt[0:128, cols],
                                        preferred_element_type=f32)
        tst[base + 1000:base + 2000] = jnp.dot(a1, wt[128:256, cols],
                                               preferred_element_type=f32)
    w = wo[...]
    q0[...] = jnp.dot(ee0[...], w[512:640, :], preferred_element_type=f32)
    q1[...] = jnp.dot(ee1[...], w[640:768, :], preferred_element_type=f32)


_k1 = pl.pallas_call(
    _k1_body,
    out_shape=[
        jax.ShapeDtypeStruct((4000, 128), f32),
        jax.ShapeDtypeStruct((4000, 128), f32),
        jax.ShapeDtypeStruct((100, 4), f32),
        jax.ShapeDtypeStruct((100, 4), f32),
    ],
)


def _add_into(a, b):
    def row(r, _):
        for j in range(8):
            sl = pl.ds(j * 16, 16)
            a[r, sl] = a[r, sl] + b[r, sl]
        return 0

    lax.fori_loop(0, CH, row, 0)


def _k2_body(ust, tst, x0h, x1h, dst, yst, rst, deg_out,
             i0f, i1f, i0c, i1c, bufa, bufb, bufc, bufd, didx_all, degbuf,
             rbuf, resbuf, deg_sh, sema, semb, semc, semd):
    c = lax.axis_index("c")
    s = lax.axis_index("s")
    tbase = c * 2000
    z16 = jnp.zeros((16,), f32)
    ones16 = jnp.ones((16,), f32)

    pltpu.sync_copy(x0h.at[pl.ds(s * 640, 640)], i0f)
    pltpu.sync_copy(x1h.at[pl.ds(s * 640, 640)], i1f)
    for r in range(40):
        sl = pl.ds(r * 16, 16)
        i0f[sl] = i0f[sl] + tbase
        i1f[sl] = i1f[sl] + (tbase + 1000)

    for k in range(5):
        off = s * 640 + k * CH
        _copy128(i0c, i0f, k * CH)
        _copy128(i1c, i1f, k * CH)
        du = pltpu.async_copy(ust.at[i0c], bufa, sema)
        dv = pltpu.async_copy(ust.at[i1c], bufb, semb)
        dt = pltpu.async_copy(tst.at[i0c], bufc, semc)
        dw = pltpu.async_copy(tst.at[i1c], bufd, semd)
        du.wait()
        dv.wait()
        _add_into(bufa, bufb)
        pltpu.sync_copy(bufa, yst.at[pl.ds(c * NP + off, CH)])
        dt.wait()
        dw.wait()
        _add_into(bufc, bufd)
        pltpu.sync_copy(bufc, rst.at[pl.ds(c * NP + off, CH)])

    def zdeg(q, _):
        degbuf[pl.ds(q * 16, 16)] = z16
        return 0

    lax.fori_loop(0, 1024, zdeg, 0)
    eph = EP // NS // 2
    pltpu.sync_copy(dst.at[pl.ds(c * (EP // 2) + s * eph, eph)], didx_all)

    def echunk(k2, _):
        for j in range(8):
            dj = didx_all[pl.ds(k2 * CH + j * 16, 16)]
            plsc.addupdate_scatter(degbuf, [dj], ones16)
        return 0

    lax.fori_loop(0, eph // CH, echunk, 0)
    pltpu.sync_copy(degbuf, deg_sh.at[s])
    plsc.subcore_barrier()
    for p in range(NS):
        pltpu.sync_copy(deg_sh.at[p, pl.ds(s * 1024, 1024)], rbuf.at[p])

    def red(g, _):
        sl = pl.ds(g * 16, 16)
        v = rbuf[0, sl]
        for p in range(1, NS):
            v = v + rbuf[p, sl]
        resbuf[sl] = v
        return 0

    lax.fori_loop(0, 64, red, 0)
    pltpu.sync_copy(resbuf, deg_out.at[pl.ds(c * 16384 + s * 1024, 1024)])


_k2 = functools.partial(
    pl.kernel,
    out_type=[
        jax.ShapeDtypeStruct((2 * NP, 128), f32),
        jax.ShapeDtypeStruct((2 * NP, 128), f32),
        jax.ShapeDtypeStruct((2 * 16384,), f32),
    ],
    mesh=plsc.VectorSubcoreMesh(**_MESH),
    compiler_params=_SC_PARAMS,
    scratch_types=[
        pltpu.VMEM((640,), i32),
        pltpu.VMEM((640,), i32),
        pltpu.VMEM((CH,), i32),
        pltpu.VMEM((CH,), i32),
        pltpu.VMEM((CH, 128), f32),
        pltpu.VMEM((CH, 128), f32),
        pltpu.VMEM((CH, 128), f32),
        pltpu.VMEM((CH, 128), f32),
        pltpu.VMEM((EP // NS // 2,), i32),
        pltpu.VMEM((16384,), f32),
        pltpu.VMEM((NS, 1024), f32),
        pltpu.VMEM((1024,), f32),
        pltpu.VMEM_SHARED((NS, 16384), f32),
        pltpu.SemaphoreType.DMA,
        pltpu.SemaphoreType.DMA,
        pltpu.SemaphoreType.DMA,
        pltpu.SemaphoreType.DMA,
    ],
)(_k2_body)


_ECH = EP // NS // CH


def _copy128(dst_ref, src_ref, base):
    for j in range(8):
        dst_ref[pl.ds(j * 16, 16)] = src_ref[pl.ds(base + j * 16, 16)]


def _msg_body(table, src, dst, acc_out,
              sidx_flat, didx_flat, sidx0, sidx1, didx0, didx1,
              rows0, rows1, zbuf, acc, sem0, sem1):
    c = lax.axis_index("c")
    s = lax.axis_index("s")
    tab_off = c * NP
    epw = EP // NS
    eph = epw // 2
    z16 = jnp.zeros((16,), f32)

    for i in range(8):
        for j in range(8):
            zbuf[i, pl.ds(j * 16, 16)] = z16

    def zacc(q, _):
        pltpu.sync_copy(zbuf, acc.at[pl.ds(s * 640 + q * 8, 8)])
        return 0

    lax.fori_loop(0, 80, zacc, 0)
    plsc.subcore_barrier()

    def fire(k, sidx, rows, sem):
        _copy128(sidx, sidx_flat, k * CH)
        pltpu.async_copy(table.at[sidx], rows, sem)

    def consume(k, sidx, didx, rows, sem):
        pltpu.make_async_copy(table.at[sidx], rows, sem).wait()
        _copy128(didx, didx_flat, k * CH)
        pltpu.sync_copy(rows, acc.at[didx], add=True)

    nck = eph // CH
    for ph in range(2):
        pltpu.sync_copy(src.at[pl.ds(s * epw + ph * eph, eph)], sidx_flat)
        pltpu.sync_copy(dst.at[pl.ds(s * epw + ph * eph, eph)], didx_flat)

        def addoff(r, _):
            sl = pl.ds(r * 16, 16)
            sidx_flat[sl] = sidx_flat[sl] + tab_off
            return 0

        lax.fori_loop(0, eph // 16, addoff, 0)
        fire(0, sidx0, rows0, sem0)

        def outer(g, _):
            fire(2 * g + 1, sidx1, rows1, sem1)
            consume(2 * g, sidx0, didx0, rows0, sem0)

            @pl.when(g < nck // 2 - 1)
            def _():
                fire(2 * g + 2, sidx0, rows0, sem0)

            consume(2 * g + 1, sidx1, didx1, rows1, sem1)
            return 0

        lax.fori_loop(0, nck // 2, outer, 0)

    plsc.subcore_barrier()
    pltpu.sync_copy(acc.at[pl.ds(s * 640, 640)],
                    acc_out.at[pl.ds(tab_off + s * 640, 640)])


_kmsg = functools.partial(
    pl.kernel,
    out_type=jax.ShapeDtypeStruct((2 * NP, 128), f32),
    mesh=plsc.VectorSubcoreMesh(**_MESH),
    compiler_params=_SC_PARAMS,
    scratch_types=[
        pltpu.VMEM((EP // NS // 2,), i32),
        pltpu.VMEM((EP // NS // 2,), i32),
        pltpu.VMEM((CH,), i32),
        pltpu.VMEM((CH,), i32),
        pltpu.VMEM((CH,), i32),
        pltpu.VMEM((CH,), i32),
        pltpu.VMEM((CH, 128), f32),
        pltpu.VMEM((CH, 128), f32),
        pltpu.VMEM((8, 128), f32),
        pltpu.VMEM_SHARED((NP, 128), f32),
        pltpu.SemaphoreType.DMA,
        pltpu.SemaphoreType.DMA,
    ],
)(_msg_body)


def _k4_body(a_ref, r_ref, d_ref, wrel, wroot, b0_ref, z_ref, xr_ref,
             dsum_ref):
    a = jnp.concatenate([a_ref[0], a_ref[1]], axis=1)
    r = jnp.concatenate([r_ref[0], r_ref[1]], axis=1)
    d = d_ref[0] + d_ref[1]
    dsum_ref[...] = d
    inv = 1.0 / jnp.maximum(d, 1.0)
    pre = a * inv + r + b0_ref[...]
    x1 = jnp.where(pre > 0, pre, jnp.exp(pre) - 1.0)
    z = jnp.dot(x1, wrel[...], preferred_element_type=f32)
    z_ref[0] = z[:, :128]
    z_ref[1] = z[:, 128:]
    xr_ref[...] = jnp.dot(x1, wroot[...], preferred_element_type=f32)


_BM = 256
_k4 = pl.pallas_call(
    _k4_body,
    grid=(NP // _BM,),
    in_specs=[
        pl.BlockSpec((2, _BM, 128), lambda i: (0, i, 0)),
        pl.BlockSpec((2, _BM, 128), lambda i: (0, i, 0)),
        pl.BlockSpec((2, _BM, 1), lambda i: (0, i, 0)),
        pl.BlockSpec((256, 256), lambda i: (0, 0)),
        pl.BlockSpec((256, 256), lambda i: (0, 0)),
        pl.BlockSpec((1, 256), lambda i: (0, 0)),
    ],
    out_specs=[
        pl.BlockSpec((2, _BM, 128), lambda i: (0, i, 0)),
        pl.BlockSpec((_BM, 256), lambda i: (i, 0)),
        pl.BlockSpec((_BM, 1), lambda i: (i, 0)),
    ],
    out_shape=[
        jax.ShapeDtypeStruct((2, NP, 128), f32),
        jax.ShapeDtypeStruct((NP, 256), f32),
        jax.ShapeDtypeStruct((NP, 1), f32),
    ],
)


def _k6_body(a_ref, xr_ref, d_ref, wcat, b1_ref, p_ref):
    a = jnp.concatenate([a_ref[0], a_ref[1]], axis=1)
    inv = 1.0 / jnp.maximum(d_ref[...], 1.0)
    pre = a * inv + xr_ref[...] + b1_ref[...]
    x2 = jnp.where(pre > 0, pre, jnp.exp(pre) - 1.0)
    p_ref[...] = jnp.dot(x2, wcat[...], preferred_element_type=f32)


_k6 = pl.pallas_call(
    _k6_body,
    grid=(NP // _BM,),
    in_specs=[
        pl.BlockSpec((2, _BM, 128), lambda i: (0, i, 0)),
        pl.BlockSpec((_BM, 256), lambda i: (i, 0)),
        pl.BlockSpec((_BM, 1), lambda i: (i, 0)),
        pl.BlockSpec((256, 8), lambda i: (0, 0)),
        pl.BlockSpec((1, 256), lambda i: (0, 0)),
    ],
    out_specs=pl.BlockSpec((_BM, 8), lambda i: (i, 0)),
    out_shape=jax.ShapeDtypeStruct((NP, 8), f32),
)


def _k7_body(st_ref, w_ref, bo_ref, s_ref):
    s_ref[...] = (jnp.dot(st_ref[...], w_ref[...],
                          preferred_element_type=f32) + bo_ref[...])


_SBM = 2000
_k7 = pl.pallas_call(
    _k7_body,
    grid=(E // _SBM,),
    in_specs=[
        pl.BlockSpec((_SBM, 16), lambda i: (i, 0)),
        pl.BlockSpec((16, 4), lambda i: (0, 0)),
        pl.BlockSpec((1, 4), lambda i: (0, 0)),
    ],
    out_specs=pl.BlockSpec((_SBM, 4), lambda i: (i, 0)),
    out_shape=jax.ShapeDtypeStruct((EP, 4), f32),
)


def _k8_body(p_hbm, q0_hbm, q1_hbm, s_hbm, src, dst, ea0, ea1, out,
             pbuf, q0b, q1b, sbuf, obuf, sia, dia, e0a, e1a):
    c = lax.axis_index("c")
    s = lax.axis_index("s")
    w = s * NC + c
    per_w = EP // (NC * NS)
    half = per_w // 2
    pltpu.sync_copy(p_hbm, pbuf)
    pltpu.sync_copy(q0_hbm, q0b)
    pltpu.sync_copy(q1_hbm, q1b)
    pltpu.sync_copy(src.at[pl.ds(w * per_w, per_w)], sia)
    pltpu.sync_copy(dst.at[pl.ds(w * per_w, per_w)], dia)
    pltpu.sync_copy(ea0.at[pl.ds(w * per_w, per_w)], e0a)
    pltpu.sync_copy(ea1.at[pl.ds(w * per_w, per_w)], e1a)
    iota16 = lax.iota(i32, 16)
    div4 = iota16 // 4
    mod4 = iota16 % 4

    for h in range(2):
        base = w * per_w + h * half
        pltpu.sync_copy(s_hbm.at[pl.ds(base * 4, half * 4)], sbuf)

        def group(g, _):
            for u in range(4):
                e = (h * half) + (g * 16 + u * 4) + div4
                sj = plsc.load_gather(sia, [e])
                dj = plsc.load_gather(dia, [e])
                a0 = plsc.load_gather(e0a, [e])
                a1 = plsc.load_gather(e1a, [e])
                sl = pl.ds(g * 64 + u * 16, 16)
                v = sbuf[sl]
                v = v + plsc.load_gather(pbuf, [sj * 8 + mod4])
                v = v + plsc.load_gather(pbuf, [dj * 8 + (mod4 + 4)])
                v = v + plsc.load_gather(q0b, [a0 * 4 + mod4])
                v = v + plsc.load_gather(q1b, [a1 * 4 + mod4])
                obuf[sl] = v
            return 0

        lax.fori_loop(0, half // 16, group, 0)
        pltpu.sync_copy(obuf, out.at[pl.ds(base * 4, half * 4)])


_k8 = functools.partial(
    pl.kernel,
    out_type=jax.ShapeDtypeStruct((EP * 4,), f32),
    mesh=plsc.VectorSubcoreMesh(**_MESH),
    compiler_params=_SC_PARAMS,
    scratch_types=[
        pltpu.VMEM((NP * 8,), f32),
        pltpu.VMEM((400,), f32),
        pltpu.VMEM((400,), f32),
        pltpu.VMEM((2560 * 4,), f32),
        pltpu.VMEM((2560 * 4,), f32),
        pltpu.VMEM((EP // 32,), i32),
        pltpu.VMEM((EP // 32,), i32),
        pltpu.VMEM((EP // 32,), i32),
        pltpu.VMEM((EP // 32,), i32),
    ],
)(_k8_body)


def kernel(x, edge_index, edge_attr, static_edge_features,
           node_emb_0, node_emb_1, edge_emb_0, edge_emb_1,
           W_rel0, W_root0, b0, W_rel1, W_root1, b1, Wo, bo):
    x0 = jnp.pad(x[:, 0].astype(i32), (0, NP - N))
    x1 = jnp.pad(x[:, 1].astype(i32), (0, NP - N))
    src = jnp.pad(edge_index[0].astype(i32), (0, EP - E))
    dst = jnp.pad(edge_index[1].astype(i32), (0, EP - E), constant_values=N)
    ea0 = jnp.pad(edge_attr[:, 0].astype(i32), (0, EP - E))
    ea1 = jnp.pad(edge_attr[:, 1].astype(i32), (0, EP - E))
    ust, tst, q0, q1 = _k1(node_emb_0, node_emb_1, edge_emb_0, edge_emb_1,
                           W_rel0, W_root0, Wo)
    yst, rst, deg = _k2(ust, tst, x0, x1, dst)
    acc1 = _kmsg(yst, src, dst)
    degh = jnp.stack([deg[:NP], deg[16384:16384 + NP]]).reshape(2, NP, 1)
    z3, xr1, deg2 = _k4(acc1.reshape(2, NP, 128), rst.reshape(2, NP, 128),
                        degh, W_rel1, W_root1, b0.reshape(1, 256))
    acc2 = _kmsg(z3.reshape(2 * NP, 128), src, dst)
    wcat = jnp.concatenate([Wo[0:256], Wo[256:512]], axis=1)
    p = _k6(acc2.reshape(2, NP, 128), xr1, deg2, wcat, b1.reshape(1, 256))
    s_edges = _k7(static_edge_features, Wo[768:784],
                  bo.reshape(1, 4))
    out = _k8(p.reshape(NP * 8), q0.reshape(400), q1.reshape(400),
              s_edges.reshape(EP * 4), src, dst, ea0, ea1)
    return out.reshape(EP, 4)[:E]


# speedup vs baseline: 1.2401x; 1.0625x over previous
import functools

import jax
import jax.numpy as jnp
from jax import lax
from jax.experimental import pallas as pl
from jax.experimental.pallas import tpu as pltpu
from jax.experimental.pallas import tpu_sc as plsc

N = 10000
E = 160000
NP = 10240
EP = 163840
NC = 2
NS = 16
CH = 128
f32 = jnp.float32
i32 = jnp.int32

_MESH = dict(core_axis_name="c", subcore_axis_name="s", num_cores=NC,
             num_subcores=NS)
_SC_PARAMS = pltpu.CompilerParams(needs_layout_passes=False)


def _k1_body(e0, e1, ee0, ee1, wrel, wroot, wo, ust, tst, q0, q1):
    a0 = e0[...]
    a1 = e1[...]
    wr = wrel[...]
    wt = wroot[...]
    for c in range(2):
        cols = slice(c * 128, (c + 1) * 128)
        base = c * 2000
        ust[base:base + 1000] = jnp.dot(a0, wr[0:128, cols],
                                        preferred_element_type=f32)
        ust[base + 1000:base + 2000] = jnp.dot(a1, wr[128:256, cols],
                                               preferred_element_type=f32)
        tst[base:base + 1000] = jnp.dot(a0, wt[0:128, cols],
                                        preferred_element_type=f32)
        tst[base + 1000:base + 2000] = jnp.dot(a1, wt[128:256, cols],
                                               preferred_element_type=f32)
    w = wo[...]
    q0[...] = jnp.dot(ee0[...], w[512:640, :], preferred_element_type=f32)
    q1[...] = jnp.dot(ee1[...], w[640:768, :], preferred_element_type=f32)


_k1 = pl.pallas_call(
    _k1_body,
    out_shape=[
        jax.ShapeDtypeStruct((4000, 128), f32),
        jax.ShapeDtypeStruct((4000, 128), f32),
        jax.ShapeDtypeStruct((100, 4), f32),
        jax.ShapeDtypeStruct((100, 4), f32),
    ],
)


def _add_into(a, b):
    def row(r, _):
        for j in range(8):
            sl = pl.ds(j * 16, 16)
            a[r, sl] = a[r, sl] + b[r, sl]
        return 0

    lax.fori_loop(0, CH, row, 0)


def _k2_body(ust, tst, x0h, x1h, dst, yst, rst, deg_out,
             i0f, i1f, i0c, i1c, bufa, bufb, bufc, bufd, didx_all, degbuf,
             rbuf, resbuf, deg_sh, sema, semb, semc, semd):
    c = lax.axis_index("c")
    s = lax.axis_index("s")
    tbase = c * 2000
    z16 = jnp.zeros((16,), f32)
    ones16 = jnp.ones((16,), f32)

    pltpu.sync_copy(x0h.at[pl.ds(s * 640, 640)], i0f)
    pltpu.sync_copy(x1h.at[pl.ds(s * 640, 640)], i1f)
    for r in range(40):
        sl = pl.ds(r * 16, 16)
        i0f[sl] = i0f[sl] + tbase
        i1f[sl] = i1f[sl] + (tbase + 1000)

    for k in range(5):
        off = s * 640 + k * CH
        _copy128(i0c, i0f, k * CH)
        _copy128(i1c, i1f, k * CH)
        du = pltpu.async_copy(ust.at[i0c], bufa, sema)
        dv = pltpu.async_copy(ust.at[i1c], bufb, semb)
        dt = pltpu.async_copy(tst.at[i0c], bufc, semc)
        dw = pltpu.async_copy(tst.at[i1c], bufd, semd)
        du.wait()
        dv.wait()
        _add_into(bufa, bufb)
        pltpu.sync_copy(bufa, yst.at[pl.ds(c * NP + off, CH)])
        dt.wait()
        dw.wait()
        _add_into(bufc, bufd)
        pltpu.sync_copy(bufc, rst.at[pl.ds(c * NP + off, CH)])

    @pl.when(c == 0)
    def _():
        def zdeg(q, _):
            degbuf[pl.ds(q * 16, 16)] = z16
            return 0

        lax.fori_loop(0, 1024, zdeg, 0)
        pltpu.sync_copy(dst.at[pl.ds(s * (EP // NS), EP // NS)], didx_all)

        def echunk(k2, _):
            for j in range(8):
                dj = didx_all[pl.ds(k2 * CH + j * 16, 16)]
                plsc.addupdate_scatter(degbuf, [dj], ones16)
            return 0

        lax.fori_loop(0, _ECH, echunk, 0)
        pltpu.sync_copy(degbuf, deg_sh.at[s])
        plsc.subcore_barrier()
        for p in range(NS):
            pltpu.sync_copy(deg_sh.at[p, pl.ds(s * 1024, 1024)], rbuf.at[p])

        def red(g, _):
            sl = pl.ds(g * 16, 16)
            v = rbuf[0, sl]
            for p in range(1, NS):
                v = v + rbuf[p, sl]
            resbuf[sl] = v
            return 0

        lax.fori_loop(0, 64, red, 0)
        pltpu.sync_copy(resbuf, deg_out.at[pl.ds(s * 1024, 1024)])


_k2 = functools.partial(
    pl.kernel,
    out_type=[
        jax.ShapeDtypeStruct((2 * NP, 128), f32),
        jax.ShapeDtypeStruct((2 * NP, 128), f32),
        jax.ShapeDtypeStruct((16384,), f32),
    ],
    mesh=plsc.VectorSubcoreMesh(**_MESH),
    compiler_params=_SC_PARAMS,
    scratch_types=[
        pltpu.VMEM((640,), i32),
        pltpu.VMEM((640,), i32),
        pltpu.VMEM((CH,), i32),
        pltpu.VMEM((CH,), i32),
        pltpu.VMEM((CH, 128), f32),
        pltpu.VMEM((CH, 128), f32),
        pltpu.VMEM((CH, 128), f32),
        pltpu.VMEM((CH, 128), f32),
        pltpu.VMEM((EP // NS,), i32),
        pltpu.VMEM((16384,), f32),
        pltpu.VMEM((NS, 1024), f32),
        pltpu.VMEM((1024,), f32),
        pltpu.VMEM_SHARED((NS, 16384), f32),
        pltpu.SemaphoreType.DMA,
        pltpu.SemaphoreType.DMA,
        pltpu.SemaphoreType.DMA,
        pltpu.SemaphoreType.DMA,
    ],
)(_k2_body)


_ECH = EP // NS // CH


def _copy128(dst_ref, src_ref, base):
    for j in range(8):
        dst_ref[pl.ds(j * 16, 16)] = src_ref[pl.ds(base + j * 16, 16)]


def _msg_body(table, src, dst, acc_out,
              sidx_flat, didx_flat, sidx0, sidx1, didx0, didx1,
              rows0, rows1, zbuf, acc, sem0, sem1):
    c = lax.axis_index("c")
    s = lax.axis_index("s")
    tab_off = c * NP
    epw = EP // NS
    eph = epw // 2
    z16 = jnp.zeros((16,), f32)

    for i in range(8):
        for j in range(8):
            zbuf[i, pl.ds(j * 16, 16)] = z16

    def zacc(q, _):
        pltpu.sync_copy(zbuf, acc.at[pl.ds(s * 640 + q * 8, 8)])
        return 0

    lax.fori_loop(0, 80, zacc, 0)
    plsc.subcore_barrier()

    def fire(k, sidx, rows, sem):
        _copy128(sidx, sidx_flat, k * CH)
        pltpu.async_copy(table.at[sidx], rows, sem)

    def consume(k, sidx, didx, rows, sem):
        pltpu.make_async_copy(table.at[sidx], rows, sem).wait()
        _copy128(didx, didx_flat, k * CH)
        pltpu.sync_copy(rows, acc.at[didx], add=True)

    nck = eph // CH
    for ph in range(2):
        pltpu.sync_copy(src.at[pl.ds(s * epw + ph * eph, eph)], sidx_flat)
        pltpu.sync_copy(dst.at[pl.ds(s * epw + ph * eph, eph)], didx_flat)

        def addoff(r, _):
            sl = pl.ds(r * 16, 16)
            sidx_flat[sl] = sidx_flat[sl] + tab_off
            return 0

        lax.fori_loop(0, eph // 16, addoff, 0)
        fire(0, sidx0, rows0, sem0)

        def outer(g, _):
            fire(2 * g + 1, sidx1, rows1, sem1)
            consume(2 * g, sidx0, didx0, rows0, sem0)

            @pl.when(g < nck // 2 - 1)
            def _():
                fire(2 * g + 2, sidx0, rows0, sem0)

            consume(2 * g + 1, sidx1, didx1, rows1, sem1)
            return 0

        lax.fori_loop(0, nck // 2, outer, 0)

    plsc.subcore_barrier()
    pltpu.sync_copy(acc.at[pl.ds(s * 640, 640)],
                    acc_out.at[pl.ds(tab_off + s * 640, 640)])


_kmsg = functools.partial(
    pl.kernel,
    out_type=jax.ShapeDtypeStruct((2 * NP, 128), f32),
    mesh=plsc.VectorSubcoreMesh(**_MESH),
    compiler_params=_SC_PARAMS,
    scratch_types=[
        pltpu.VMEM((EP // NS // 2,), i32),
        pltpu.VMEM((EP // NS // 2,), i32),
        pltpu.VMEM((CH,), i32),
        pltpu.VMEM((CH,), i32),
        pltpu.VMEM((CH,), i32),
        pltpu.VMEM((CH,), i32),
        pltpu.VMEM((CH, 128), f32),
        pltpu.VMEM((CH, 128), f32),
        pltpu.VMEM((8, 128), f32),
        pltpu.VMEM_SHARED((NP, 128), f32),
        pltpu.SemaphoreType.DMA,
        pltpu.SemaphoreType.DMA,
    ],
)(_msg_body)


def _k4_body(a_ref, r_ref, d_ref, wrel, wroot, b0_ref, z_ref, xr_ref):
    a = jnp.concatenate([a_ref[0], a_ref[1]], axis=1)
    r = jnp.concatenate([r_ref[0], r_ref[1]], axis=1)
    inv = 1.0 / jnp.maximum(d_ref[...], 1.0)
    pre = a * inv + r + b0_ref[...]
    x1 = jnp.where(pre > 0, pre, jnp.exp(pre) - 1.0)
    z = jnp.dot(x1, wrel[...], preferred_element_type=f32)
    z_ref[0] = z[:, :128]
    z_ref[1] = z[:, 128:]
    xr_ref[...] = jnp.dot(x1, wroot[...], preferred_element_type=f32)


_BM = 256
_k4 = pl.pallas_call(
    _k4_body,
    grid=(NP // _BM,),
    in_specs=[
        pl.BlockSpec((2, _BM, 128), lambda i: (0, i, 0)),
        pl.BlockSpec((2, _BM, 128), lambda i: (0, i, 0)),
        pl.BlockSpec((_BM, 1), lambda i: (i, 0)),
        pl.BlockSpec((256, 256), lambda i: (0, 0)),
        pl.BlockSpec((256, 256), lambda i: (0, 0)),
        pl.BlockSpec((1, 256), lambda i: (0, 0)),
    ],
    out_specs=[
        pl.BlockSpec((2, _BM, 128), lambda i: (0, i, 0)),
        pl.BlockSpec((_BM, 256), lambda i: (i, 0)),
    ],
    out_shape=[
        jax.ShapeDtypeStruct((2, NP, 128), f32),
        jax.ShapeDtypeStruct((NP, 256), f32),
    ],
)


def _k6_body(a_ref, xr_ref, d_ref, wcat, b1_ref, p_ref):
    a = jnp.concatenate([a_ref[0], a_ref[1]], axis=1)
    inv = 1.0 / jnp.maximum(d_ref[...], 1.0)
    pre = a * inv + xr_ref[...] + b1_ref[...]
    x2 = jnp.where(pre > 0, pre, jnp.exp(pre) - 1.0)
    p_ref[...] = jnp.dot(x2, wcat[...], preferred_element_type=f32)


_k6 = pl.pallas_call(
    _k6_body,
    grid=(NP // _BM,),
    in_specs=[
        pl.BlockSpec((2, _BM, 128), lambda i: (0, i, 0)),
        pl.BlockSpec((_BM, 256), lambda i: (i, 0)),
        pl.BlockSpec((_BM, 1), lambda i: (i, 0)),
        pl.BlockSpec((256, 8), lambda i: (0, 0)),
        pl.BlockSpec((1, 256), lambda i: (0, 0)),
    ],
    out_specs=pl.BlockSpec((_BM, 8), lambda i: (i, 0)),
    out_shape=jax.ShapeDtypeStruct((NP, 8), f32),
)


def _k7_body(st_ref, w_ref, bo_ref, s_ref):
    s_ref[...] = (jnp.dot(st_ref[...], w_ref[...],
                          preferred_element_type=f32) + bo_ref[...])


_SBM = 2000
_k7 = pl.pallas_call(
    _k7_body,
    grid=(E // _SBM,),
    in_specs=[
        pl.BlockSpec((_SBM, 16), lambda i: (i, 0)),
        pl.BlockSpec((16, 4), lambda i: (0, 0)),
        pl.BlockSpec((1, 4), lambda i: (0, 0)),
    ],
    out_specs=pl.BlockSpec((_SBM, 4), lambda i: (i, 0)),
    out_shape=jax.ShapeDtypeStruct((EP, 4), f32),
)


def _k8_body(p_hbm, q0_hbm, q1_hbm, s_hbm, src, dst, ea0, ea1, out,
             pbuf, q0b, q1b, sbuf, obuf, sia, dia, e0a, e1a):
    c = lax.axis_index("c")
    s = lax.axis_index("s")
    w = s * NC + c
    per_w = EP // (NC * NS)
    half = per_w // 2
    pltpu.sync_copy(p_hbm, pbuf)
    pltpu.sync_copy(q0_hbm, q0b)
    pltpu.sync_copy(q1_hbm, q1b)
    pltpu.sync_copy(src.at[pl.ds(w * per_w, per_w)], sia)
    pltpu.sync_copy(dst.at[pl.ds(w * per_w, per_w)], dia)
    pltpu.sync_copy(ea0.at[pl.ds(w * per_w, per_w)], e0a)
    pltpu.sync_copy(ea1.at[pl.ds(w * per_w, per_w)], e1a)
    iota16 = lax.iota(i32, 16)
    div4 = iota16 // 4
    mod4 = iota16 % 4

    for h in range(2):
        base = w * per_w + h * half
        pltpu.sync_copy(s_hbm.at[pl.ds(base * 4, half * 4)], sbuf)

        def group(g, _):
            for u in range(4):
                e = (h * half) + (g * 16 + u * 4) + div4
                sj = plsc.load_gather(sia, [e])
                dj = plsc.load_gather(dia, [e])
                a0 = plsc.load_gather(e0a, [e])
                a1 = plsc.load_gather(e1a, [e])
                sl = pl.ds(g * 64 + u * 16, 16)
                v = sbuf[sl]
                v = v + plsc.load_gather(pbuf, [sj * 8 + mod4])
                v = v + plsc.load_gather(pbuf, [dj * 8 + (mod4 + 4)])
                v = v + plsc.load_gather(q0b, [a0 * 4 + mod4])
                v = v + plsc.load_gather(q1b, [a1 * 4 + mod4])
                obuf[sl] = v
            return 0

        lax.fori_loop(0, half // 16, group, 0)
        pltpu.sync_copy(obuf, out.at[pl.ds(base * 4, half * 4)])


_k8 = functools.partial(
    pl.kernel,
    out_type=jax.ShapeDtypeStruct((EP * 4,), f32),
    mesh=plsc.VectorSubcoreMesh(**_MESH),
    compiler_params=_SC_PARAMS,
    scratch_types=[
        pltpu.VMEM((NP * 8,), f32),
        pltpu.VMEM((400,), f32),
        pltpu.VMEM((400,), f32),
        pltpu.VMEM((2560 * 4,), f32),
        pltpu.VMEM((2560 * 4,), f32),
        pltpu.VMEM((EP // 32,), i32),
        pltpu.VMEM((EP // 32,), i32),
        pltpu.VMEM((EP // 32,), i32),
        pltpu.VMEM((EP // 32,), i32),
    ],
)(_k8_body)


def kernel(x, edge_index, edge_attr, static_edge_features,
           node_emb_0, node_emb_1, edge_emb_0, edge_emb_1,
           W_rel0, W_root0, b0, W_rel1, W_root1, b1, Wo, bo):
    x0 = jnp.pad(x[:, 0].astype(i32), (0, NP - N))
    x1 = jnp.pad(x[:, 1].astype(i32), (0, NP - N))
    src = jnp.pad(edge_index[0].astype(i32), (0, EP - E))
    dst = jnp.pad(edge_index[1].astype(i32), (0, EP - E), constant_values=N)
    ea0 = jnp.pad(edge_attr[:, 0].astype(i32), (0, EP - E))
    ea1 = jnp.pad(edge_attr[:, 1].astype(i32), (0, EP - E))
    ust, tst, q0, q1 = _k1(node_emb_0, node_emb_1, edge_emb_0, edge_emb_1,
                           W_rel0, W_root0, Wo)
    yst, rst, deg = _k2(ust, tst, x0, x1, dst)
    acc1 = _kmsg(yst, src, dst)
    deg2 = deg[:NP].reshape(NP, 1)
    z3, xr1 = _k4(acc1.reshape(2, NP, 128), rst.reshape(2, NP, 128), deg2,
                  W_rel1, W_root1, b0.reshape(1, 256))
    acc2 = _kmsg(z3.reshape(2 * NP, 128), src, dst)
    wcat = jnp.concatenate([Wo[0:256], Wo[256:512]], axis=1)
    p = _k6(acc2.reshape(2, NP, 128), xr1, deg2, wcat, b1.reshape(1, 256))
    s_edges = _k7(static_edge_features, Wo[768:784],
                  bo.reshape(1, 4))
    out = _k8(p.reshape(NP * 8), q0.reshape(400), q1.reshape(400),
              s_edges.reshape(EP * 4), src, dst, ea0, ea1)
    return out.reshape(EP, 4)[:E]
